# baseline (speedup 1.0000x reference)
import functools

import jax
import jax.numpy as jnp
from jax.experimental import pallas as pl
from jax.experimental.pallas import tpu as pltpu

N, D, H, L, K, T = 8192, 1024, 512, 64, 16, 50
DF = 1.0
BLK = 2048


def _body(x_ref, w1_ref, b1_ref, w2_ref, b2_ref, decw_ref, decb_ref,
          swz_ref, swx_ref, sb_ref, c_ref,
          z_ref, q_ref, surv_ref, xhat_ref, rec_ref):
    x = x_ref[...]
    xb = x.astype(jnp.bfloat16)
    h1 = jnp.maximum(
        jnp.dot(xb, w1_ref[...].astype(jnp.bfloat16),
                preferred_element_type=jnp.float32) + b1_ref[...], 0.0)
    z = jnp.dot(h1.astype(jnp.bfloat16), w2_ref[...].astype(jnp.bfloat16),
                preferred_element_type=jnp.float32) + b2_ref[...]
    z_ref[...] = z
    zb = z.astype(jnp.bfloat16)

    c = c_ref[...]
    zc = jax.lax.dot_general(z, c, (((1,), (1,)), ((), ())),
                             preferred_element_type=jnp.float32)
    z2 = jnp.sum(z * z, axis=1, keepdims=True)
    c2 = jnp.sum(c * c, axis=1)[None, :]
    dist2 = jnp.maximum(z2 - 2.0 * zc + c2, 0.0)
    logits = -0.5 * (DF + 1.0) * jnp.log1p(dist2 / DF)
    logits = logits - jnp.max(logits, axis=1, keepdims=True)
    e = jnp.exp(logits)
    q_ref[...] = e / jnp.sum(e, axis=1, keepdims=True)

    surv_ref[...] = (
        jnp.dot(zb, swz_ref[...].astype(jnp.bfloat16), preferred_element_type=jnp.float32)
        + jnp.dot(xb, swx_ref[...].astype(jnp.bfloat16), preferred_element_type=jnp.float32)
        + sb_ref[...])

    x_hat = jnp.dot(zb, decw_ref[...].astype(jnp.bfloat16), preferred_element_type=jnp.float32) + decb_ref[...]
    xhat_ref[...] = x_hat
    d = x_hat - x
    rec_ref[...] = jnp.sum(d * d, axis=1, keepdims=True) * (1.0 / D)


@jax.jit
def kernel(x, enc_W1, enc_b1, enc_W2, enc_b2, dec_W, dec_b, surv_W, surv_b, centers):
    grid = (N // BLK,)
    full = lambda shape: pl.BlockSpec(shape, lambda i: (0,) * len(shape))
    row = lambda w: pl.BlockSpec((BLK, w), lambda i: (i, 0))

    z, q, surv, x_hat, rec = pl.pallas_call(
        _body,
        grid=grid,
        in_specs=[
            row(D),
            full((D, H)), full((1, H)),
            full((H, L)), full((1, L)),
            full((L, D)), full((1, D)),
            full((L, T)), full((D, T)), full((1, T)),
            full((K, L)),
        ],
        out_specs=[row(L), row(K), row(T), row(D), row(1)],
        out_shape=[
            jax.ShapeDtypeStruct((N, L), jnp.float32),
            jax.ShapeDtypeStruct((N, K), jnp.float32),
            jax.ShapeDtypeStruct((N, T), jnp.float32),
            jax.ShapeDtypeStruct((N, D), jnp.float32),
            jax.ShapeDtypeStruct((N, 1), jnp.float32),
        ],
        compiler_params=pltpu.CompilerParams(
            dimension_semantics=("arbitrary",)),
    )(x, enc_W1, enc_b1[None, :], enc_W2, enc_b2[None, :],
      dec_W, dec_b[None, :], surv_W[:L], surv_W[L:], surv_b[None, :],
      centers)

    zeros_nl = jnp.zeros((N, L), jnp.float32)
    kld = jnp.zeros((N,), jnp.float32)
    return (z, zeros_nl, zeros_nl, kld, x_hat, rec[:, 0], q, surv, centers)
